# baseline (speedup 1.0000x reference)
import functools

import jax
import jax.numpy as jnp
from jax import lax
from jax.experimental import pallas as pl
from jax.experimental.pallas import tpu as pltpu
from jax.experimental.pallas import tpu_sc as plsc

_info = plsc.get_sparse_core_info()
_NC, _NS = _info.num_cores, _info.num_subcores
_NW = _NC * _NS

_CH = 32
_NBUF = 2


def _make_sc_broadcast(B, T, D, dtype):
    rows_per_w = T // _NW
    chunks = rows_per_w // _CH
    mesh = plsc.VectorSubcoreMesh(core_axis_name="c", subcore_axis_name="s")

    @functools.partial(
        pl.kernel,
        mesh=mesh,
        out_type=jax.ShapeDtypeStruct((B, T, D), dtype),
        scratch_types=[
            pltpu.VMEM((_NBUF, _CH, D), dtype),
            pltpu.SemaphoreType.DMA,
            pltpu.SemaphoreType.DMA,
        ],
    )
    def sc_broadcast(table_hbm, out_hbm, buf, rsem, wsem):
        wid = lax.axis_index("s") * _NC + lax.axis_index("c")
        base = wid * rows_per_w

        def rd(i):
            r0 = base + i * _CH
            return pltpu.make_async_copy(
                table_hbm.at[pl.ds(r0, _CH)], buf.at[i % _NBUF], rsem
            )

        def wr(i, b):
            r0 = base + i * _CH
            return pltpu.make_async_copy(
                buf.at[i % _NBUF], out_hbm.at[b, pl.ds(r0, _CH)], wsem
            )

        rd(0).start()
        for i in range(chunks):
            rd(i).wait()
            if i >= _NBUF - 1:
                for b in range(B):
                    wr(i - (_NBUF - 1), b).wait()
            if i + 1 < chunks:
                rd(i + 1).start()
            for b in range(B):
                wr(i, b).start()
        for i in range(chunks - _NBUF + 1, chunks):
            for b in range(B):
                wr(i, b).wait()

    return sc_broadcast


def kernel(x, pos_table):
    B, T, D = x.shape
    return _make_sc_broadcast(B, T, D, pos_table.dtype)(pos_table)


# speedup vs baseline: 1.6467x; 1.6467x over previous
import jax
import jax.numpy as jnp
from jax.experimental import pallas as pl
from jax.experimental.pallas import tpu as pltpu

BT = 1024


def _body(table_ref, out_ref, sem):
    t = pl.program_id(0)
    B = out_ref.shape[0]
    for b in range(B):
        pltpu.make_async_copy(
            table_ref, out_ref.at[b, pl.ds(t * BT, BT)], sem
        ).start()
    for b in range(B):
        pltpu.make_async_copy(
            table_ref, out_ref.at[b, pl.ds(t * BT, BT)], sem
        ).wait()


def kernel(x, pos_table):
    B, T, D = x.shape
    return pl.pallas_call(
        _body,
        grid=(T // BT,),
        in_specs=[pl.BlockSpec((BT, D), lambda t: (t, 0))],
        out_specs=pl.BlockSpec(memory_space=pltpu.HBM),
        out_shape=jax.ShapeDtypeStruct((B, T, D), pos_table.dtype),
        scratch_shapes=[pltpu.SemaphoreType.DMA],
    )(pos_table)


# speedup vs baseline: 1.7564x; 1.0666x over previous
import jax
import jax.numpy as jnp
from jax.experimental import pallas as pl
from jax.experimental.pallas import tpu as pltpu

CH = 512
NB = 4


def _body(table_hbm, out_hbm, buf, rsem, wsem):
    B = out_hbm.shape[0]
    T = table_hbm.shape[0]
    nc = T // CH

    def rd(c):
        return pltpu.make_async_copy(
            table_hbm.at[pl.ds(c * CH, CH)], buf.at[c % NB], rsem
        )

    def wr(c, b):
        return pltpu.make_async_copy(
            buf.at[c % NB], out_hbm.at[b, pl.ds(c * CH, CH)], wsem
        )

    rd(0).start()
    rd(1).start()
    for c in range(nc):
        rd(c).wait()
        if c >= 2:
            for b in range(B):
                wr(c - 2, b).wait()
        if c + 2 < nc:
            rd(c + 2).start()
        for b in range(B):
            wr(c, b).start()
    for c in range(nc - 2, nc):
        for b in range(B):
            wr(c, b).wait()


def kernel(x, pos_table):
    B, T, D = x.shape
    return pl.pallas_call(
        _body,
        in_specs=[pl.BlockSpec(memory_space=pltpu.HBM)],
        out_specs=pl.BlockSpec(memory_space=pltpu.HBM),
        out_shape=jax.ShapeDtypeStruct((B, T, D), pos_table.dtype),
        scratch_shapes=[
            pltpu.VMEM((NB, CH, D), pos_table.dtype),
            pltpu.SemaphoreType.DMA,
            pltpu.SemaphoreType.DMA,
        ],
    )(pos_table)
